# baseline (speedup 1.0000x reference)
import jax
import jax.numpy as jnp
from jax import lax
from jax.experimental import pallas as pl
from jax.experimental.pallas import tpu as pltpu
from jax.experimental.pallas import tpu_sc as plsc

N = 10000
E = 160000
D = 128
G = 8
H = 128

NC = 2
NS = 16
NW = NC * NS
CH = 128

EH = E // 2
NCH_H = EH // CH
NPT = NCH_H // NW
NEXTRA = NCH_H - NPT * NW

BLK_N = 2000
BLK_E = 4000
NBE_H = EH // BLK_E

ZROWS = 640
NBUF = 3


def _prep_body(x_ref, b3_ref, u_ref, wa_ref, wb_ref, wd_ref, be1_ref,
               a2_ref, bm_ref):
    ug = jnp.dot(u_ref[...], wd_ref[...], preferred_element_type=jnp.float32)
    ug = ug + be1_ref[...]
    brow = b3_ref[0]
    iota = lax.broadcasted_iota(jnp.int32, (G, BLK_N), 0).astype(jnp.float32)
    oht = (iota == brow).astype(jnp.float32)
    ugb = lax.dot_general(oht, ug, (((0,), (0,)), ((), ())),
                          preferred_element_type=jnp.float32)
    x = x_ref[...]
    a2_ref[...] = jnp.dot(x, wa_ref[...], preferred_element_type=jnp.float32) + ugb
    bm_ref[...] = jnp.dot(x, wb_ref[...], preferred_element_type=jnp.float32)


def _run_prep(x, batch3, u, wa, wb, wd, be1):
    nsteps = N // BLK_N
    return pl.pallas_call(
        _prep_body,
        grid=(nsteps,),
        in_specs=[
            pl.BlockSpec((BLK_N, D), lambda i: (i, 0)),
            pl.BlockSpec((1, 1, BLK_N), lambda i: (i, 0, 0)),
            pl.BlockSpec((G, D), lambda i: (0, 0)),
            pl.BlockSpec((D, H), lambda i: (0, 0)),
            pl.BlockSpec((D, H), lambda i: (0, 0)),
            pl.BlockSpec((D, H), lambda i: (0, 0)),
            pl.BlockSpec((1, H), lambda i: (0, 0)),
        ],
        out_specs=[
            pl.BlockSpec((BLK_N, H), lambda i: (i, 0)),
            pl.BlockSpec((BLK_N, H), lambda i: (i, 0)),
        ],
        out_shape=[
            jax.ShapeDtypeStruct((N, H), jnp.float32),
            jax.ShapeDtypeStruct((N, H), jnp.float32),
        ],
    )(x, batch3, u, wa, wb, wd, be1)


def _sc_gather_body(a2, bm, batch_h, src3, dst3, srcx, dstx, ga, gb, bs,
                    src_all, dst_all, rowa0, rowa1, rowa2, rowb0, rowb1, rowb2,
                    bsv0, bsv1, bsv2, batch_v, sga0, sga1, sga2,
                    sgb0, sgb1, sgb2, swa0, swa1, swa2,
                    swb0, swb1, swb2, sws0, sws1, sws2):
    c = lax.axis_index("c")
    s = lax.axis_index("s")
    w = s * NC + c
    lo = w * NPT

    rowa = (rowa0, rowa1, rowa2)
    rowb = (rowb0, rowb1, rowb2)
    bsv = (bsv0, bsv1, bsv2)
    sga = (sga0, sga1, sga2)
    sgb = (sgb0, sgb1, sgb2)
    swa = (swa0, swa1, swa2)
    swb = (swb0, swb1, swb2)
    sws = (sws0, sws1, sws2)

    pltpu.sync_copy(src3.at[w], src_all)
    pltpu.sync_copy(dst3.at[w], dst_all)
    pltpu.sync_copy(batch_h, batch_v)

    def compute_bs(i, b):
        for k in range(CH // 16):
            idx = src_all[i, pl.ds(k * 16, 16)]
            vals = plsc.load_gather(batch_v, [idx])
            bsv[b][pl.ds(k * 16, 16)] = vals.astype(jnp.float32)

    gd = {}
    wd = {}

    def start(i, b):
        if i - NBUF in wd:
            for d in wd.pop(i - NBUF):
                d.wait()
        gd[i] = (
            pltpu.async_copy(a2.at[src_all.at[i]], rowa[b], sga[b]),
            pltpu.async_copy(bm.at[dst_all.at[i]], rowb[b], sgb[b]),
        )

    def finish(i, b):
        da, db = gd.pop(i)
        compute_bs(i, b)
        da.wait()
        db.wait()
        base = pl.multiple_of((lo + i) * CH, CH)
        wd[i] = (
            pltpu.async_copy(rowa[b], ga.at[pl.ds(base, CH)], swa[b]),
            pltpu.async_copy(rowb[b], gb.at[pl.ds(base, CH)], swb[b]),
            pltpu.async_copy(bsv[b], bs.at[pl.ds(base, CH)], sws[b]),
        )

    start(0, 0)
    start(1, 1)
    for i in range(2, NPT + 2):
        if i < NPT:
            start(i, i % NBUF)
        finish(i - 2, (i - 2) % NBUF)
    for ds_ in wd.values():
        for d in ds_:
            d.wait()
    wd.clear()

    @pl.when(w < NEXTRA)
    def _():
        j = NPT * NW + w
        pltpu.sync_copy(srcx.at[w], src_all.at[pl.ds(0, 1)])
        pltpu.sync_copy(dstx.at[w], dst_all.at[pl.ds(0, 1)])
        da = pltpu.async_copy(a2.at[src_all.at[0]], rowa[0], sga[0])
        db = pltpu.async_copy(bm.at[dst_all.at[0]], rowb[0], sgb[0])
        compute_bs(0, 0)
        da.wait()
        db.wait()
        base = pl.multiple_of(j * CH, CH)
        pltpu.sync_copy(rowa[0], ga.at[pl.ds(base, CH)])
        pltpu.sync_copy(rowb[0], gb.at[pl.ds(base, CH)])
        pltpu.sync_copy(bsv[0], bs.at[pl.ds(base, CH)])


def _run_sc_gather(a2, bm, batch_i, src3, dst3, srcx, dstx):
    mesh = plsc.VectorSubcoreMesh(core_axis_name="c", subcore_axis_name="s",
                                  num_cores=NC, num_subcores=NS)
    fn = pl.kernel(
        _sc_gather_body,
        compiler_params=pltpu.CompilerParams(needs_layout_passes=False),
        out_type=(
            jax.ShapeDtypeStruct((EH, H), jnp.float32),
            jax.ShapeDtypeStruct((EH, H), jnp.float32),
            jax.ShapeDtypeStruct((EH,), jnp.float32),
        ),
        mesh=mesh,
        scratch_types=[
            pltpu.VMEM((NPT, CH), jnp.int32),
            pltpu.VMEM((NPT, CH), jnp.int32),
        ] + [pltpu.VMEM((CH, H), jnp.float32)] * 6
          + [pltpu.VMEM((CH,), jnp.float32)] * 3
          + [pltpu.VMEM((N,), jnp.int32)]
          + [pltpu.SemaphoreType.DMA] * 15,
    )
    return fn(a2, bm, batch_i, src3, dst3, srcx, dstx)


def _edge_math(ea_ref, ga_ref, gb_ref, bs3_ref, wc_ref, w2_ref, be2_ref):
    ea = ea_ref[...]
    pre = jnp.dot(ea, wc_ref[...], preferred_element_type=jnp.float32)
    pre = pre + ga_ref[...] + gb_ref[...]
    h = jnp.maximum(pre, 0.0)
    out = ea + jnp.dot(h, w2_ref[...], preferred_element_type=jnp.float32)
    out = out + be2_ref[...]
    brow = bs3_ref[0]
    iota = lax.broadcasted_iota(jnp.int32, (G, BLK_E), 0).astype(jnp.float32)
    oht = (iota == brow).astype(jnp.float32)
    part = jnp.dot(oht, out, preferred_element_type=jnp.float32)
    return out, part


def _edge_body_a(ea_ref, ga_ref, gb_ref, bs3_ref, wc_ref, w2_ref, be2_ref,
                 out_ref, dup_ref, eagg_ref):
    i = pl.program_id(0)
    out, part = _edge_math(ea_ref, ga_ref, gb_ref, bs3_ref, wc_ref, w2_ref,
                           be2_ref)
    out_ref[...] = out
    dup_ref[...] = out

    @pl.when(i == 0)
    def _():
        eagg_ref[...] = jnp.zeros_like(eagg_ref)

    eagg_ref[...] += part


def _edge_body_b(ea_ref, ga_ref, gb_ref, bs3_ref, wc_ref, w2_ref, be2_ref,
                 xprev_ref, out_ref, eagg_ref):
    del xprev_ref
    i = pl.program_id(0)
    out, part = _edge_math(ea_ref, ga_ref, gb_ref, bs3_ref, wc_ref, w2_ref,
                           be2_ref)
    out_ref[...] = out

    @pl.when(i == 0)
    def _():
        eagg_ref[...] = jnp.zeros_like(eagg_ref)

    eagg_ref[...] += part


def _run_edge_a(ea_full, ga, gb, bs3, wc, w2, be2):
    return pl.pallas_call(
        _edge_body_a,
        grid=(NBE_H,),
        in_specs=[
            pl.BlockSpec((BLK_E, D), lambda i: (i, 0)),
            pl.BlockSpec((BLK_E, H), lambda i: (i, 0)),
            pl.BlockSpec((BLK_E, H), lambda i: (i, 0)),
            pl.BlockSpec((1, 1, BLK_E), lambda i: (i, 0, 0)),
            pl.BlockSpec((D, H), lambda i: (0, 0)),
            pl.BlockSpec((H, D), lambda i: (0, 0)),
            pl.BlockSpec((1, D), lambda i: (0, 0)),
        ],
        out_specs=[
            pl.BlockSpec((BLK_E, D), lambda i: (i, 0)),
            pl.BlockSpec((BLK_E, D), lambda i: (i, 0)),
            pl.BlockSpec((G, D), lambda i: (0, 0)),
        ],
        out_shape=[
            jax.ShapeDtypeStruct((E, D), jnp.float32),
            jax.ShapeDtypeStruct((EH, D), jnp.float32),
            jax.ShapeDtypeStruct((G, D), jnp.float32),
        ],
    )(ea_full, ga, gb, bs3, wc, w2, be2)


def _run_edge_b(ea_full, ga, gb, bs3, wc, w2, be2, xprev):
    off = NBE_H
    return pl.pallas_call(
        _edge_body_b,
        grid=(NBE_H,),
        in_specs=[
            pl.BlockSpec((BLK_E, D), lambda i: (i + off, 0)),
            pl.BlockSpec((BLK_E, H), lambda i: (i, 0)),
            pl.BlockSpec((BLK_E, H), lambda i: (i, 0)),
            pl.BlockSpec((1, 1, BLK_E), lambda i: (i, 0, 0)),
            pl.BlockSpec((D, H), lambda i: (0, 0)),
            pl.BlockSpec((H, D), lambda i: (0, 0)),
            pl.BlockSpec((1, D), lambda i: (0, 0)),
            pl.BlockSpec(memory_space=pl.ANY),
        ],
        out_specs=[
            pl.BlockSpec((BLK_E, D), lambda i: (i + off, 0)),
            pl.BlockSpec((G, D), lambda i: (0, 0)),
        ],
        out_shape=[
            jax.ShapeDtypeStruct((E, D), jnp.float32),
            jax.ShapeDtypeStruct((G, D), jnp.float32),
        ],
        input_output_aliases={7: 0},
    )(ea_full, ga, gb, bs3, wc, w2, be2, xprev)


def _sc_scatter_body(coff, enew, dst3, dstx, zin, parts,
                     dst_all, rows0, rows1, acc, sl0, sl1, sa0, sa1):
    c = lax.axis_index("c")
    s = lax.axis_index("s")
    w = s * NC + c
    lo = w * NPT
    z0 = pl.multiple_of(s * ZROWS, 8)

    rows = (rows0, rows1)
    sl = (sl0, sl1)
    sa = (sa0, sa1)

    @pl.when(s < NS - 1)
    def _():
        pltpu.sync_copy(zin, acc.at[pl.ds(z0, ZROWS)])

    @pl.when(s == NS - 1)
    def _():
        last = N - (NS - 1) * ZROWS
        pltpu.sync_copy(zin.at[pl.ds(0, last)],
                        acc.at[pl.ds((NS - 1) * ZROWS, last)])

    pltpu.sync_copy(dst3.at[w], dst_all)
    plsc.subcore_barrier()

    ld = {}
    ad = {}

    def load(i, b):
        if i - 2 in ad:
            ad.pop(i - 2).wait()
        ld[i] = pltpu.async_copy(
            enew.at[pl.ds(pl.multiple_of((coff + lo + i) * CH, CH), CH)],
            rows[b], sl[b])

    def add(i, b):
        ld.pop(i).wait()
        ad[i] = pltpu.async_copy(rows[b], acc.at[dst_all.at[i]], sa[b],
                                 add=True)

    load(0, 0)
    for i in range(1, NPT + 1):
        if i < NPT:
            load(i, i % 2)
        add(i - 1, (i - 1) % 2)
    for d in ad.values():
        d.wait()
    ad.clear()

    @pl.when(w < NEXTRA)
    def _():
        j = coff + NPT * NW + w
        pltpu.sync_copy(dstx.at[w], dst_all.at[pl.ds(0, 1)])
        pltpu.sync_copy(enew.at[pl.ds(pl.multiple_of(j * CH, CH), CH)],
                        rows[0])
        pltpu.sync_copy(rows[0], acc.at[dst_all.at[0]], add=True)

    plsc.subcore_barrier()

    @pl.when(s < NS - 1)
    def _():
        pltpu.sync_copy(acc.at[pl.ds(z0, ZROWS)],
                        parts.at[c].at[pl.ds(z0, ZROWS)])

    @pl.when(s == NS - 1)
    def _():
        last = N - (NS - 1) * ZROWS
        pltpu.sync_copy(acc.at[pl.ds((NS - 1) * ZROWS, last)],
                        parts.at[c].at[pl.ds((NS - 1) * ZROWS, last)])


def _run_sc_scatter(enew, dst3, dstx, zin, coff):
    mesh = plsc.VectorSubcoreMesh(core_axis_name="c", subcore_axis_name="s",
                                  num_cores=NC, num_subcores=NS)

    def body(enew, dst3, dstx, zin, parts, dst_all, rows0, rows1, acc,
             sl0, sl1, sa0, sa1):
        _sc_scatter_body(coff, enew, dst3, dstx, zin, parts, dst_all,
                         rows0, rows1, acc, sl0, sl1, sa0, sa1)

    fn = pl.kernel(
        body,
        out_type=jax.ShapeDtypeStruct((NC, N, H), jnp.float32),
        mesh=mesh,
        scratch_types=[
            pltpu.VMEM((NPT, CH), jnp.int32),
            pltpu.VMEM((CH, H), jnp.float32),
            pltpu.VMEM((CH, H), jnp.float32),
            pltpu.VMEM_SHARED((N, H), jnp.float32),
            pltpu.SemaphoreType.DMA,
            pltpu.SemaphoreType.DMA,
            pltpu.SemaphoreType.DMA,
            pltpu.SemaphoreType.DMA,
        ],
    )
    return fn(enew, dst3, dstx, zin)


def _node_body(x_ref, p0_ref, p1_ref, p2_ref, p3_ref, b3_ref,
               ea_ref, eb_ref, u_ref,
               wna_ref, wnb_ref, wnc_ref, bn1_ref, wn2_ref, bn2_ref,
               wga_ref, wgb_ref, wgc_ref, bg1_ref, wg2_ref, bg2_ref,
               xn_ref, un_ref, nagg_ref):
    i = pl.program_id(0)
    nsteps = pl.num_programs(0)

    u = u_ref[...]
    ugn = jnp.dot(u, wnc_ref[...], preferred_element_type=jnp.float32)
    ugn = ugn + bn1_ref[...]
    brow = b3_ref[0]
    iota = lax.broadcasted_iota(jnp.int32, (G, BLK_N), 0).astype(jnp.float32)
    oht = (iota == brow).astype(jnp.float32)
    ugb = lax.dot_general(oht, ugn, (((0,), (0,)), ((), ())),
                          preferred_element_type=jnp.float32)

    x = x_ref[...]
    agg = (p0_ref[0] + p1_ref[0]) + (p2_ref[0] + p3_ref[0])
    pre = jnp.dot(x, wna_ref[...], preferred_element_type=jnp.float32)
    pre = pre + jnp.dot(agg, wnb_ref[...], preferred_element_type=jnp.float32)
    pre = pre + ugb
    h = jnp.maximum(pre, 0.0)
    xn = x + jnp.dot(h, wn2_ref[...], preferred_element_type=jnp.float32)
    xn = xn + bn2_ref[...]
    xn_ref[...] = xn

    part = jnp.dot(oht, xn, preferred_element_type=jnp.float32)

    @pl.when(i == 0)
    def _():
        nagg_ref[...] = jnp.zeros_like(nagg_ref)

    nagg_ref[...] += part

    @pl.when(i == nsteps - 1)
    def _():
        nagg = nagg_ref[...]
        eagg = ea_ref[...] + eb_ref[...]
        gpre = jnp.dot(nagg, wga_ref[...], preferred_element_type=jnp.float32)
        gpre = gpre + jnp.dot(eagg, wgb_ref[...],
                              preferred_element_type=jnp.float32)
        gpre = gpre + jnp.dot(u, wgc_ref[...], preferred_element_type=jnp.float32)
        gpre = gpre + bg1_ref[...]
        gh = jnp.maximum(gpre, 0.0)
        un = u + jnp.dot(gh, wg2_ref[...], preferred_element_type=jnp.float32)
        un_ref[...] = un + bg2_ref[...]


def _run_node(x, pa, pb, batch3, eagga, eaggb, u,
              wna, wnb, wnc, bn1, wn2, bn2,
              wga, wgb, wgc, bg1, wg2, bg2):
    nsteps = N // BLK_N
    full = lambda r, c: pl.BlockSpec((r, c), lambda i: (0, 0))
    return pl.pallas_call(
        _node_body,
        grid=(nsteps,),
        in_specs=[
            pl.BlockSpec((BLK_N, D), lambda i: (i, 0)),
            pl.BlockSpec((1, BLK_N, H), lambda i: (0, i, 0)),
            pl.BlockSpec((1, BLK_N, H), lambda i: (1, i, 0)),
            pl.BlockSpec((1, BLK_N, H), lambda i: (0, i, 0)),
            pl.BlockSpec((1, BLK_N, H), lambda i: (1, i, 0)),
            pl.BlockSpec((1, 1, BLK_N), lambda i: (i, 0, 0)),
            full(G, D), full(G, D), full(G, D),
            full(D, H), full(D, H), full(D, H), full(1, H),
            full(H, D), full(1, D),
            full(D, H), full(D, H), full(D, H), full(1, H),
            full(H, D), full(1, D),
        ],
        out_specs=[
            pl.BlockSpec((BLK_N, D), lambda i: (i, 0)),
            pl.BlockSpec((G, D), lambda i: (0, 0)),
        ],
        out_shape=[
            jax.ShapeDtypeStruct((N, D), jnp.float32),
            jax.ShapeDtypeStruct((G, D), jnp.float32),
        ],
        scratch_shapes=[pltpu.VMEM((G, D), jnp.float32)],
    )(x, pa, pa, pb, pb, batch3, eagga, eaggb, u,
      wna, wnb, wnc, bn1, wn2, bn2,
      wga, wgb, wgc, bg1, wg2, bg2)


def kernel(x, edge_attr, edge_index, batch, u, We1, be1, We2, be2,
           Wn1, bn1, Wn2, bn2, Wg1, bg1, Wg2, bg2):
    src = edge_index[0].astype(jnp.int32)
    dst = edge_index[1].astype(jnp.int32)
    batch_i = batch.astype(jnp.int32)

    nslab = NPT * NW
    src2 = src.reshape(E // CH, CH)
    dst2 = dst.reshape(E // CH, CH)

    def half_views(arr2, h):
        o = h * NCH_H
        slab = arr2[o:o + nslab].reshape(NW, NPT, CH)
        extra = arr2[o + nslab:o + NCH_H].reshape(NEXTRA, 1, CH)
        return slab, extra

    src3a, srcxa = half_views(src2, 0)
    dst3a, dstxa = half_views(dst2, 0)
    src3b, srcxb = half_views(src2, 1)
    dst3b, dstxb = half_views(dst2, 1)

    batch3 = batch_i.astype(jnp.float32).reshape(N // BLK_N, 1, BLK_N)

    wa = We1[:D]
    wb = We1[D:2 * D]
    wc = We1[2 * D:3 * D]
    wd = We1[3 * D:]
    be1r = be1.reshape(1, H)
    be2r = be2.reshape(1, D)
    wna, wnb, wnc = Wn1[:D], Wn1[D:2 * D], Wn1[2 * D:]
    bn1r = bn1.reshape(1, H)
    bn2r = bn2.reshape(1, D)
    wga, wgb, wgc = Wg1[:D], Wg1[D:2 * D], Wg1[2 * D:]
    bg1r = bg1.reshape(1, H)
    bg2r = bg2.reshape(1, D)
    zin = jnp.zeros((ZROWS, H), jnp.float32)

    a2, bm = _run_prep(x, batch3, u, wa, wb, wd, be1r)

    gaa, gba, bsa = _run_sc_gather(a2, bm, batch_i, src3a, dst3a, srcxa, dstxa)
    gab, gbb, bsb = _run_sc_gather(a2, bm, batch_i, src3b, dst3b, srcxb, dstxb)

    bs3a = bsa.reshape(NBE_H, 1, BLK_E)
    bs3b = bsb.reshape(NBE_H, 1, BLK_E)

    xout, dupa, eagga = _run_edge_a(edge_attr, gaa, gba, bs3a, wc, We2, be2r)
    parts_a = _run_sc_scatter(dupa, dst3a, dstxa, zin, 0)
    edge_new, eaggb = _run_edge_b(edge_attr, gab, gbb, bs3b, wc, We2, be2r,
                                  xout)
    parts_b = _run_sc_scatter(edge_new, dst3b, dstxb, zin, NCH_H)

    x_new, u_new = _run_node(
        x, parts_a, parts_b, batch3, eagga, eaggb, u,
        wna, wnb, wnc, bn1r, Wn2, bn2r,
        wga, wgb, wgc, bg1r, Wg2, bg2r)

    return (x_new, edge_new, u_new)


# speedup vs baseline: 1.2128x; 1.2128x over previous
import jax
import jax.numpy as jnp
from jax import lax
from jax.experimental import pallas as pl
from jax.experimental.pallas import tpu as pltpu
from jax.experimental.pallas import tpu_sc as plsc

N = 10000
E = 160000
D = 128
G = 8
H = 128

NC = 2
NS = 16
NW = NC * NS
CH = 128

EH = E // 2
NCH_H = EH // CH
NPT = NCH_H // NW
NEXTRA = NCH_H - NPT * NW

BLK_N = 2000
BLK_E = 8000
NBE_H = EH // BLK_E

ZROWS = 640
NBUF = 3

NCHUNK = E // CH
NPT_S = NCHUNK // NW
NEXTRA_S = NCHUNK - NPT_S * NW


def _prep_body(x_ref, b3_ref, u_ref, wa_ref, wb_ref, wd_ref, be1_ref,
               a2_ref, bm_ref):
    ug = jnp.dot(u_ref[...], wd_ref[...], preferred_element_type=jnp.float32)
    ug = ug + be1_ref[...]
    brow = b3_ref[0]
    iota = lax.broadcasted_iota(jnp.int32, (G, BLK_N), 0).astype(jnp.float32)
    oht = (iota == brow).astype(jnp.float32)
    ugb = lax.dot_general(oht, ug, (((0,), (0,)), ((), ())),
                          preferred_element_type=jnp.float32)
    x = x_ref[...]
    a2_ref[...] = jnp.dot(x, wa_ref[...], preferred_element_type=jnp.float32) + ugb
    bm_ref[...] = jnp.dot(x, wb_ref[...], preferred_element_type=jnp.float32)


def _run_prep(x, batch3, u, wa, wb, wd, be1):
    nsteps = N // BLK_N
    return pl.pallas_call(
        _prep_body,
        grid=(nsteps,),
        in_specs=[
            pl.BlockSpec((BLK_N, D), lambda i: (i, 0)),
            pl.BlockSpec((1, 1, BLK_N), lambda i: (i, 0, 0)),
            pl.BlockSpec((G, D), lambda i: (0, 0)),
            pl.BlockSpec((D, H), lambda i: (0, 0)),
            pl.BlockSpec((D, H), lambda i: (0, 0)),
            pl.BlockSpec((D, H), lambda i: (0, 0)),
            pl.BlockSpec((1, H), lambda i: (0, 0)),
        ],
        out_specs=[
            pl.BlockSpec((BLK_N, H), lambda i: (i, 0)),
            pl.BlockSpec((BLK_N, H), lambda i: (i, 0)),
        ],
        out_shape=[
            jax.ShapeDtypeStruct((N, H), jnp.float32),
            jax.ShapeDtypeStruct((N, H), jnp.float32),
        ],
    )(x, batch3, u, wa, wb, wd, be1)


def _sc_gather_body(a2, bm, batch_h, src3, dst3, srcx, dstx, ga, bs,
                    src_all, dst_all, rowa0, rowa1, rowa2,
                    bsv0, bsv1, bsv2, batch_v, sga0, sga1, sga2,
                    sgb0, sgb1, sgb2, swa0, swa1, swa2, sws0, sws1, sws2):
    c = lax.axis_index("c")
    s = lax.axis_index("s")
    w = s * NC + c
    lo = w * NPT_S

    rowa = (rowa0, rowa1, rowa2)
    bsv = (bsv0, bsv1, bsv2)
    sga = (sga0, sga1, sga2)
    sgb = (sgb0, sgb1, sgb2)
    swa = (swa0, swa1, swa2)
    sws = (sws0, sws1, sws2)

    pltpu.sync_copy(src3.at[w], src_all)
    pltpu.sync_copy(dst3.at[w], dst_all)
    pltpu.sync_copy(batch_h, batch_v)

    def compute_bs(i, b):
        for k in range(CH // 16):
            idx = src_all[i, pl.ds(k * 16, 16)]
            vals = plsc.load_gather(batch_v, [idx])
            bsv[b][pl.ds(k * 16, 16)] = vals.astype(jnp.float32)

    gd = {}
    md = {}
    wd = {}

    def start(i, b):
        if i - NBUF in wd:
            for d in wd.pop(i - NBUF):
                d.wait()
        gd[i] = pltpu.async_copy(a2.at[src_all.at[i]], rowa[b], sga[b])

    def mid(i, b):
        gd.pop(i).wait()
        md[i] = pltpu.async_copy(bm.at[dst_all.at[i]], rowa[b], sgb[b],
                                 add=True)

    def finish(i, b):
        compute_bs(i, b)
        md.pop(i).wait()
        base = pl.multiple_of((lo + i) * CH, CH)
        wd[i] = (
            pltpu.async_copy(rowa[b], ga.at[pl.ds(base, CH)], swa[b]),
            pltpu.async_copy(bsv[b], bs.at[pl.ds(base, CH)], sws[b]),
        )

    start(0, 0)
    start(1, 1)
    mid(0, 0)
    for i in range(2, NPT_S + 2):
        if i < NPT_S:
            start(i, i % NBUF)
        if i - 1 < NPT_S:
            mid(i - 1, (i - 1) % NBUF)
        finish(i - 2, (i - 2) % NBUF)
    for ds_ in wd.values():
        for d in ds_:
            d.wait()
    wd.clear()

    @pl.when(w < NEXTRA_S)
    def _():
        j = NPT_S * NW + w
        pltpu.sync_copy(srcx.at[w], src_all.at[pl.ds(0, 1)])
        pltpu.sync_copy(dstx.at[w], dst_all.at[pl.ds(0, 1)])
        da = pltpu.async_copy(a2.at[src_all.at[0]], rowa[0], sga[0])
        compute_bs(0, 0)
        da.wait()
        db = pltpu.async_copy(bm.at[dst_all.at[0]], rowa[0], sgb[0],
                              add=True)
        db.wait()
        base = pl.multiple_of(j * CH, CH)
        pltpu.sync_copy(rowa[0], ga.at[pl.ds(base, CH)])
        pltpu.sync_copy(bsv[0], bs.at[pl.ds(base, CH)])


def _run_sc_gather(a2, bm, batch_i, src3, dst3, srcx, dstx):
    mesh = plsc.VectorSubcoreMesh(core_axis_name="c", subcore_axis_name="s",
                                  num_cores=NC, num_subcores=NS)
    fn = pl.kernel(
        _sc_gather_body,
        compiler_params=pltpu.CompilerParams(needs_layout_passes=False),
        out_type=(
            jax.ShapeDtypeStruct((E, H), jnp.float32),
            jax.ShapeDtypeStruct((E,), jnp.float32),
        ),
        mesh=mesh,
        scratch_types=[
            pltpu.VMEM((NPT_S, CH), jnp.int32),
            pltpu.VMEM((NPT_S, CH), jnp.int32),
        ] + [pltpu.VMEM((CH, H), jnp.float32)] * 3
          + [pltpu.VMEM((CH,), jnp.float32)] * 3
          + [pltpu.VMEM((N,), jnp.int32)]
          + [pltpu.SemaphoreType.DMA] * 12,
    )
    return fn(a2, bm, batch_i, src3, dst3, srcx, dstx)


def _edge_math(ea_ref, g_ref, bs3_ref, wc_ref, w2_ref, be2_ref):
    ea = ea_ref[...]
    pre = jnp.dot(ea, wc_ref[...], preferred_element_type=jnp.float32)
    pre = pre + g_ref[...]
    h = jnp.maximum(pre, 0.0)
    out = ea + jnp.dot(h, w2_ref[...], preferred_element_type=jnp.float32)
    out = out + be2_ref[...]
    brow = bs3_ref[0]
    iota = lax.broadcasted_iota(jnp.int32, (G, BLK_E), 0).astype(jnp.float32)
    oht = (iota == brow).astype(jnp.float32)
    part = jnp.dot(oht, out, preferred_element_type=jnp.float32)
    return out, part


def _edge_body(ea_ref, g_ref, bs3_ref, wc_ref, w2_ref, be2_ref,
               out_ref, eagg_ref):
    i = pl.program_id(0)
    out, part = _edge_math(ea_ref, g_ref, bs3_ref, wc_ref, w2_ref, be2_ref)
    out_ref[...] = out

    @pl.when(i == 0)
    def _():
        eagg_ref[...] = jnp.zeros_like(eagg_ref)

    eagg_ref[...] += part


def _run_edge(ea_full, g, bs3, wc, w2, be2):
    return pl.pallas_call(
        _edge_body,
        grid=(E // BLK_E,),
        in_specs=[
            pl.BlockSpec((BLK_E, D), lambda i: (i, 0)),
            pl.BlockSpec((BLK_E, H), lambda i: (i, 0)),
            pl.BlockSpec((1, 1, BLK_E), lambda i: (i, 0, 0)),
            pl.BlockSpec((D, H), lambda i: (0, 0)),
            pl.BlockSpec((H, D), lambda i: (0, 0)),
            pl.BlockSpec((1, D), lambda i: (0, 0)),
        ],
        out_specs=[
            pl.BlockSpec((BLK_E, D), lambda i: (i, 0)),
            pl.BlockSpec((G, D), lambda i: (0, 0)),
        ],
        out_shape=[
            jax.ShapeDtypeStruct((E, D), jnp.float32),
            jax.ShapeDtypeStruct((G, D), jnp.float32),
        ],
    )(ea_full, g, bs3, wc, w2, be2)


def _sc_scatter_body(enew, dst3, dstx, zin, parts,
                     dst_all, rows0, rows1, acc, sl0, sl1, sa0, sa1):
    c = lax.axis_index("c")
    s = lax.axis_index("s")
    w = s * NC + c
    lo = w * NPT_S
    z0 = pl.multiple_of(s * ZROWS, 8)

    rows = (rows0, rows1)
    sl = (sl0, sl1)
    sa = (sa0, sa1)

    @pl.when(s < NS - 1)
    def _():
        pltpu.sync_copy(zin, acc.at[pl.ds(z0, ZROWS)])

    @pl.when(s == NS - 1)
    def _():
        last = N - (NS - 1) * ZROWS
        pltpu.sync_copy(zin.at[pl.ds(0, last)],
                        acc.at[pl.ds((NS - 1) * ZROWS, last)])

    pltpu.sync_copy(dst3.at[w], dst_all)
    plsc.subcore_barrier()

    ld = {}
    ad = {}

    def load(i, b):
        if i - 2 in ad:
            ad.pop(i - 2).wait()
        ld[i] = pltpu.async_copy(
            enew.at[pl.ds(pl.multiple_of((lo + i) * CH, CH), CH)],
            rows[b], sl[b])

    def add(i, b):
        ld.pop(i).wait()
        ad[i] = pltpu.async_copy(rows[b], acc.at[dst_all.at[i]], sa[b],
                                 add=True)

    load(0, 0)
    for i in range(1, NPT_S + 1):
        if i < NPT_S:
            load(i, i % 2)
        add(i - 1, (i - 1) % 2)
    for d in ad.values():
        d.wait()
    ad.clear()

    @pl.when(w < NEXTRA_S)
    def _():
        j = NPT_S * NW + w
        pltpu.sync_copy(dstx.at[w], dst_all.at[pl.ds(0, 1)])
        pltpu.sync_copy(enew.at[pl.ds(pl.multiple_of(j * CH, CH), CH)],
                        rows[0])
        pltpu.sync_copy(rows[0], acc.at[dst_all.at[0]], add=True)

    plsc.subcore_barrier()

    @pl.when(s < NS - 1)
    def _():
        pltpu.sync_copy(acc.at[pl.ds(z0, ZROWS)],
                        parts.at[c].at[pl.ds(z0, ZROWS)])

    @pl.when(s == NS - 1)
    def _():
        last = N - (NS - 1) * ZROWS
        pltpu.sync_copy(acc.at[pl.ds((NS - 1) * ZROWS, last)],
                        parts.at[c].at[pl.ds((NS - 1) * ZROWS, last)])


def _run_sc_scatter(enew, dst3, dstx, zin):
    mesh = plsc.VectorSubcoreMesh(core_axis_name="c", subcore_axis_name="s",
                                  num_cores=NC, num_subcores=NS)

    fn = pl.kernel(
        _sc_scatter_body,
        out_type=jax.ShapeDtypeStruct((NC, N, H), jnp.float32),
        mesh=mesh,
        scratch_types=[
            pltpu.VMEM((NPT_S, CH), jnp.int32),
            pltpu.VMEM((CH, H), jnp.float32),
            pltpu.VMEM((CH, H), jnp.float32),
            pltpu.VMEM_SHARED((N, H), jnp.float32),
        ] + [pltpu.SemaphoreType.DMA] * 4,
    )
    return fn(enew, dst3, dstx, zin)


def _node_body(x_ref, p0_ref, p1_ref, b3_ref,
               ea_ref, u_ref,
               wna_ref, wnb_ref, wnc_ref, bn1_ref, wn2_ref, bn2_ref,
               wga_ref, wgb_ref, wgc_ref, bg1_ref, wg2_ref, bg2_ref,
               xn_ref, un_ref, nagg_ref):
    i = pl.program_id(0)
    nsteps = pl.num_programs(0)

    u = u_ref[...]
    ugn = jnp.dot(u, wnc_ref[...], preferred_element_type=jnp.float32)
    ugn = ugn + bn1_ref[...]
    brow = b3_ref[0]
    iota = lax.broadcasted_iota(jnp.int32, (G, BLK_N), 0).astype(jnp.float32)
    oht = (iota == brow).astype(jnp.float32)
    ugb = lax.dot_general(oht, ugn, (((0,), (0,)), ((), ())),
                          preferred_element_type=jnp.float32)

    x = x_ref[...]
    agg = p0_ref[0] + p1_ref[0]
    pre = jnp.dot(x, wna_ref[...], preferred_element_type=jnp.float32)
    pre = pre + jnp.dot(agg, wnb_ref[...], preferred_element_type=jnp.float32)
    pre = pre + ugb
    h = jnp.maximum(pre, 0.0)
    xn = x + jnp.dot(h, wn2_ref[...], preferred_element_type=jnp.float32)
    xn = xn + bn2_ref[...]
    xn_ref[...] = xn

    part = jnp.dot(oht, xn, preferred_element_type=jnp.float32)

    @pl.when(i == 0)
    def _():
        nagg_ref[...] = jnp.zeros_like(nagg_ref)

    nagg_ref[...] += part

    @pl.when(i == nsteps - 1)
    def _():
        nagg = nagg_ref[...]
        eagg = ea_ref[...]
        gpre = jnp.dot(nagg, wga_ref[...], preferred_element_type=jnp.float32)
        gpre = gpre + jnp.dot(eagg, wgb_ref[...],
                              preferred_element_type=jnp.float32)
        gpre = gpre + jnp.dot(u, wgc_ref[...], preferred_element_type=jnp.float32)
        gpre = gpre + bg1_ref[...]
        gh = jnp.maximum(gpre, 0.0)
        un = u + jnp.dot(gh, wg2_ref[...], preferred_element_type=jnp.float32)
        un_ref[...] = un + bg2_ref[...]


def _run_node(x, pa, batch3, eagg, u,
              wna, wnb, wnc, bn1, wn2, bn2,
              wga, wgb, wgc, bg1, wg2, bg2):
    nsteps = N // BLK_N
    full = lambda r, c: pl.BlockSpec((r, c), lambda i: (0, 0))
    return pl.pallas_call(
        _node_body,
        grid=(nsteps,),
        in_specs=[
            pl.BlockSpec((BLK_N, D), lambda i: (i, 0)),
            pl.BlockSpec((1, BLK_N, H), lambda i: (0, i, 0)),
            pl.BlockSpec((1, BLK_N, H), lambda i: (1, i, 0)),
            pl.BlockSpec((1, 1, BLK_N), lambda i: (i, 0, 0)),
            full(G, D), full(G, D),
            full(D, H), full(D, H), full(D, H), full(1, H),
            full(H, D), full(1, D),
            full(D, H), full(D, H), full(D, H), full(1, H),
            full(H, D), full(1, D),
        ],
        out_specs=[
            pl.BlockSpec((BLK_N, D), lambda i: (i, 0)),
            pl.BlockSpec((G, D), lambda i: (0, 0)),
        ],
        out_shape=[
            jax.ShapeDtypeStruct((N, D), jnp.float32),
            jax.ShapeDtypeStruct((G, D), jnp.float32),
        ],
        scratch_shapes=[pltpu.VMEM((G, D), jnp.float32)],
    )(x, pa, pa, batch3, eagg, u,
      wna, wnb, wnc, bn1, wn2, bn2,
      wga, wgb, wgc, bg1, wg2, bg2)


def kernel(x, edge_attr, edge_index, batch, u, We1, be1, We2, be2,
           Wn1, bn1, Wn2, bn2, Wg1, bg1, Wg2, bg2):
    src = edge_index[0].astype(jnp.int32)
    dst = edge_index[1].astype(jnp.int32)
    batch_i = batch.astype(jnp.int32)

    src2 = src.reshape(E // CH, CH)
    dst2 = dst.reshape(E // CH, CH)
    nslab_s = NPT_S * NW
    src3f = src2[:nslab_s].reshape(NW, NPT_S, CH)
    srcxf = src2[nslab_s:].reshape(NEXTRA_S, 1, CH)
    dst3f = dst2[:nslab_s].reshape(NW, NPT_S, CH)
    dstxf = dst2[nslab_s:].reshape(NEXTRA_S, 1, CH)

    batch3 = batch_i.astype(jnp.float32).reshape(N // BLK_N, 1, BLK_N)

    wa = We1[:D]
    wb = We1[D:2 * D]
    wc = We1[2 * D:3 * D]
    wd = We1[3 * D:]
    be1r = be1.reshape(1, H)
    be2r = be2.reshape(1, D)
    wna, wnb, wnc = Wn1[:D], Wn1[D:2 * D], Wn1[2 * D:]
    bn1r = bn1.reshape(1, H)
    bn2r = bn2.reshape(1, D)
    wga, wgb, wgc = Wg1[:D], Wg1[D:2 * D], Wg1[2 * D:]
    bg1r = bg1.reshape(1, H)
    bg2r = bg2.reshape(1, D)
    zin = jnp.zeros((ZROWS, H), jnp.float32)

    a2, bm = _run_prep(x, batch3, u, wa, wb, wd, be1r)

    g, bs = _run_sc_gather(a2, bm, batch_i, src3f, dst3f, srcxf, dstxf)

    bs3 = bs.reshape(E // BLK_E, 1, BLK_E)

    edge_new, eagg = _run_edge(edge_attr, g, bs3, wc, We2, be2r)

    parts = _run_sc_scatter(edge_new, dst3f, dstxf, zin)

    x_new, u_new = _run_node(
        x, parts, batch3, eagg, u,
        wna, wnb, wnc, bn1r, Wn2, bn2r,
        wga, wgb, wgc, bg1r, Wg2, bg2r)

    return (x_new, edge_new, u_new)


# speedup vs baseline: 1.2140x; 1.0010x over previous
import jax
import jax.numpy as jnp
from jax import lax
from jax.experimental import pallas as pl
from jax.experimental.pallas import tpu as pltpu
from jax.experimental.pallas import tpu_sc as plsc

N = 10000
E = 160000
D = 128
G = 8
H = 128

NC = 2
NS = 16
NW = NC * NS
CH = 128

BLK_N = 2000
BLK_E = 8000

ZROWS = 640
NBUF = 3

NCHUNK = E // CH
NPT_S = NCHUNK // NW
NEXTRA_S = NCHUNK - NPT_S * NW


def _prep_body(x_ref, b3_ref, u_ref, wa_ref, wb_ref, wd_ref, be1_ref,
               a2_ref, bm_ref):
    ug = jnp.dot(u_ref[...], wd_ref[...], preferred_element_type=jnp.float32)
    ug = ug + be1_ref[...]
    brow = b3_ref[0]
    iota = lax.broadcasted_iota(jnp.int32, (G, BLK_N), 0).astype(jnp.float32)
    oht = (iota == brow).astype(jnp.float32)
    ugb = lax.dot_general(oht, ug, (((0,), (0,)), ((), ())),
                          preferred_element_type=jnp.float32)
    x = x_ref[...]
    a2_ref[...] = jnp.dot(x, wa_ref[...], preferred_element_type=jnp.float32) + ugb
    bm_ref[...] = jnp.dot(x, wb_ref[...], preferred_element_type=jnp.float32)


def _run_prep(x, batch3, u, wa, wb, wd, be1):
    nsteps = N // BLK_N
    return pl.pallas_call(
        _prep_body,
        grid=(nsteps,),
        in_specs=[
            pl.BlockSpec((BLK_N, D), lambda i: (i, 0)),
            pl.BlockSpec((1, 1, BLK_N), lambda i: (i, 0, 0)),
            pl.BlockSpec((G, D), lambda i: (0, 0)),
            pl.BlockSpec((D, H), lambda i: (0, 0)),
            pl.BlockSpec((D, H), lambda i: (0, 0)),
            pl.BlockSpec((D, H), lambda i: (0, 0)),
            pl.BlockSpec((1, H), lambda i: (0, 0)),
        ],
        out_specs=[
            pl.BlockSpec((BLK_N, H), lambda i: (i, 0)),
            pl.BlockSpec((BLK_N, H), lambda i: (i, 0)),
        ],
        out_shape=[
            jax.ShapeDtypeStruct((N, H), jnp.float32),
            jax.ShapeDtypeStruct((N, H), jnp.float32),
        ],
    )(x, batch3, u, wa, wb, wd, be1)


def _sc_gather_body(a2, bm, batch_h, src3, dst3, srcx, dstx, ga, bs,
                    src_all, dst_all, rowa0, rowa1, rowa2,
                    bsv0, bsv1, bsv2, batch_v, sga0, sga1, sga2,
                    sgb0, sgb1, sgb2, swa0, swa1, swa2, sws0, sws1, sws2):
    c = lax.axis_index("c")
    s = lax.axis_index("s")
    w = s * NC + c
    lo = w * NPT_S

    rowa = (rowa0, rowa1, rowa2)
    bsv = (bsv0, bsv1, bsv2)
    sga = (sga0, sga1, sga2)
    sgb = (sgb0, sgb1, sgb2)
    swa = (swa0, swa1, swa2)
    sws = (sws0, sws1, sws2)

    pltpu.sync_copy(src3.at[w], src_all)
    pltpu.sync_copy(dst3.at[w], dst_all)
    pltpu.sync_copy(batch_h, batch_v)

    def compute_bs(i, b):
        for k in range(CH // 16):
            idx = src_all[i, pl.ds(k * 16, 16)]
            vals = plsc.load_gather(batch_v, [idx])
            bsv[b][pl.ds(k * 16, 16)] = vals.astype(jnp.float32)

    gd = {}
    md = {}
    wd = {}

    def start(i, b):
        if i - NBUF in wd:
            for d in wd.pop(i - NBUF):
                d.wait()
        gd[i] = pltpu.async_copy(a2.at[src_all.at[i]], rowa[b], sga[b])

    def mid(i, b):
        gd.pop(i).wait()
        md[i] = pltpu.async_copy(bm.at[dst_all.at[i]], rowa[b], sgb[b],
                                 add=True)

    def finish(i, b):
        compute_bs(i, b)
        md.pop(i).wait()
        base = pl.multiple_of((lo + i) * CH, CH)
        wd[i] = (
            pltpu.async_copy(rowa[b], ga.at[pl.ds(base, CH)], swa[b]),
            pltpu.async_copy(bsv[b], bs.at[pl.ds(base, CH)], sws[b]),
        )

    start(0, 0)
    start(1, 1)
    mid(0, 0)
    for i in range(2, NPT_S + 2):
        if i < NPT_S:
            start(i, i % NBUF)
        if i - 1 < NPT_S:
            mid(i - 1, (i - 1) % NBUF)
        finish(i - 2, (i - 2) % NBUF)
    for ds_ in wd.values():
        for d in ds_:
            d.wait()
    wd.clear()

    @pl.when(w < NEXTRA_S)
    def _():
        j = NPT_S * NW + w
        pltpu.sync_copy(srcx.at[w], src_all.at[pl.ds(0, 1)])
        pltpu.sync_copy(dstx.at[w], dst_all.at[pl.ds(0, 1)])
        da = pltpu.async_copy(a2.at[src_all.at[0]], rowa[0], sga[0])
        compute_bs(0, 0)
        da.wait()
        db = pltpu.async_copy(bm.at[dst_all.at[0]], rowa[0], sgb[0],
                              add=True)
        db.wait()
        base = pl.multiple_of(j * CH, CH)
        pltpu.sync_copy(rowa[0], ga.at[pl.ds(base, CH)])
        pltpu.sync_copy(bsv[0], bs.at[pl.ds(base, CH)])


def _run_sc_gather(a2, bm, batch_i, src3, dst3, srcx, dstx):
    mesh = plsc.VectorSubcoreMesh(core_axis_name="c", subcore_axis_name="s",
                                  num_cores=NC, num_subcores=NS)
    fn = pl.kernel(
        _sc_gather_body,
        compiler_params=pltpu.CompilerParams(needs_layout_passes=False),
        out_type=(
            jax.ShapeDtypeStruct((E, H), jnp.float32),
            jax.ShapeDtypeStruct((E,), jnp.float32),
        ),
        mesh=mesh,
        scratch_types=[
            pltpu.VMEM((NPT_S, CH), jnp.int32),
            pltpu.VMEM((NPT_S, CH), jnp.int32),
        ] + [pltpu.VMEM((CH, H), jnp.float32)] * 3
          + [pltpu.VMEM((CH,), jnp.float32)] * 3
          + [pltpu.VMEM((N,), jnp.int32)]
          + [pltpu.SemaphoreType.DMA] * 12,
    )
    return fn(a2, bm, batch_i, src3, dst3, srcx, dstx)


def _edge_math(ea_ref, g_ref, bs3_ref, wc_ref, w2_ref, be2_ref):
    ea = ea_ref[...]
    pre = jnp.dot(ea, wc_ref[...], preferred_element_type=jnp.float32)
    pre = pre + g_ref[...]
    h = jnp.maximum(pre, 0.0)
    out = ea + jnp.dot(h, w2_ref[...], preferred_element_type=jnp.float32)
    out = out + be2_ref[...]
    brow = bs3_ref[0]
    iota = lax.broadcasted_iota(jnp.int32, (G, BLK_E), 0).astype(jnp.float32)
    oht = (iota == brow).astype(jnp.float32)
    part = jnp.dot(oht, out, preferred_element_type=jnp.float32)
    return out, part


def _edge_body(ea_ref, g_ref, bs3_ref, wc_ref, w2_ref, be2_ref,
               out_ref, eagg_ref):
    i = pl.program_id(0)
    out, part = _edge_math(ea_ref, g_ref, bs3_ref, wc_ref, w2_ref, be2_ref)
    out_ref[...] = out

    @pl.when(i == 0)
    def _():
        eagg_ref[...] = jnp.zeros_like(eagg_ref)

    eagg_ref[...] += part


def _run_edge(ea_full, g, bs3, wc, w2, be2):
    return pl.pallas_call(
        _edge_body,
        grid=(E // BLK_E,),
        in_specs=[
            pl.BlockSpec((BLK_E, D), lambda i: (i, 0)),
            pl.BlockSpec((BLK_E, H), lambda i: (i, 0)),
            pl.BlockSpec((1, 1, BLK_E), lambda i: (i, 0, 0)),
            pl.BlockSpec((D, H), lambda i: (0, 0)),
            pl.BlockSpec((H, D), lambda i: (0, 0)),
            pl.BlockSpec((1, D), lambda i: (0, 0)),
        ],
        out_specs=[
            pl.BlockSpec((BLK_E, D), lambda i: (i, 0)),
            pl.BlockSpec((G, D), lambda i: (0, 0)),
        ],
        out_shape=[
            jax.ShapeDtypeStruct((E, D), jnp.float32),
            jax.ShapeDtypeStruct((G, D), jnp.float32),
        ],
    )(ea_full, g, bs3, wc, w2, be2)


def _sc_scatter_body(enew, dst3, dstx, zin, parts,
                     dst_all, rows0, rows1, acc, sl0, sl1, sa0, sa1):
    c = lax.axis_index("c")
    s = lax.axis_index("s")
    w = s * NC + c
    lo = w * NPT_S
    z0 = pl.multiple_of(s * ZROWS, 8)

    rows = (rows0, rows1)
    sl = (sl0, sl1)
    sa = (sa0, sa1)

    @pl.when(s < NS - 1)
    def _():
        pltpu.sync_copy(zin, acc.at[pl.ds(z0, ZROWS)])

    @pl.when(s == NS - 1)
    def _():
        last = N - (NS - 1) * ZROWS
        pltpu.sync_copy(zin.at[pl.ds(0, last)],
                        acc.at[pl.ds((NS - 1) * ZROWS, last)])

    pltpu.sync_copy(dst3.at[w], dst_all)
    plsc.subcore_barrier()

    ld = {}
    ad = {}

    def load(i, b):
        if i - 2 in ad:
            ad.pop(i - 2).wait()
        ld[i] = pltpu.async_copy(
            enew.at[pl.ds(pl.multiple_of((lo + i) * CH, CH), CH)],
            rows[b], sl[b])

    def add(i, b):
        ld.pop(i).wait()
        ad[i] = pltpu.async_copy(rows[b], acc.at[dst_all.at[i]], sa[b],
                                 add=True)

    load(0, 0)
    for i in range(1, NPT_S + 1):
        if i < NPT_S:
            load(i, i % 2)
        add(i - 1, (i - 1) % 2)
    for d in ad.values():
        d.wait()
    ad.clear()

    @pl.when(w < NEXTRA_S)
    def _():
        j = NPT_S * NW + w
        pltpu.sync_copy(dstx.at[w], dst_all.at[pl.ds(0, 1)])
        pltpu.sync_copy(enew.at[pl.ds(pl.multiple_of(j * CH, CH), CH)],
                        rows[0])
        pltpu.sync_copy(rows[0], acc.at[dst_all.at[0]], add=True)

    plsc.subcore_barrier()

    @pl.when(s < NS - 1)
    def _():
        pltpu.sync_copy(acc.at[pl.ds(z0, ZROWS)],
                        parts.at[c].at[pl.ds(z0, ZROWS)])

    @pl.when(s == NS - 1)
    def _():
        last = N - (NS - 1) * ZROWS
        pltpu.sync_copy(acc.at[pl.ds((NS - 1) * ZROWS, last)],
                        parts.at[c].at[pl.ds((NS - 1) * ZROWS, last)])


def _run_sc_scatter(enew, dst3, dstx, zin):
    mesh = plsc.VectorSubcoreMesh(core_axis_name="c", subcore_axis_name="s",
                                  num_cores=NC, num_subcores=NS)

    fn = pl.kernel(
        _sc_scatter_body,
        out_type=jax.ShapeDtypeStruct((NC, N, H), jnp.float32),
        mesh=mesh,
        scratch_types=[
            pltpu.VMEM((NPT_S, CH), jnp.int32),
            pltpu.VMEM((CH, H), jnp.float32),
            pltpu.VMEM((CH, H), jnp.float32),
            pltpu.VMEM_SHARED((N, H), jnp.float32),
        ] + [pltpu.SemaphoreType.DMA] * 4,
    )
    return fn(enew, dst3, dstx, zin)


def _node_body(x_ref, p0_ref, p1_ref, b3_ref,
               ea_ref, u_ref,
               wna_ref, wnb_ref, wnc_ref, bn1_ref, wn2_ref, bn2_ref,
               wga_ref, wgb_ref, wgc_ref, bg1_ref, wg2_ref, bg2_ref,
               xn_ref, un_ref, nagg_ref):
    i = pl.program_id(0)
    nsteps = pl.num_programs(0)

    u = u_ref[...]
    ugn = jnp.dot(u, wnc_ref[...], preferred_element_type=jnp.float32)
    ugn = ugn + bn1_ref[...]
    brow = b3_ref[0]
    iota = lax.broadcasted_iota(jnp.int32, (G, BLK_N), 0).astype(jnp.float32)
    oht = (iota == brow).astype(jnp.float32)
    ugb = lax.dot_general(oht, ugn, (((0,), (0,)), ((), ())),
                          preferred_element_type=jnp.float32)

    x = x_ref[...]
    agg = p0_ref[0] + p1_ref[0]
    pre = jnp.dot(x, wna_ref[...], preferred_element_type=jnp.float32)
    pre = pre + jnp.dot(agg, wnb_ref[...], preferred_element_type=jnp.float32)
    pre = pre + ugb
    h = jnp.maximum(pre, 0.0)
    xn = x + jnp.dot(h, wn2_ref[...], preferred_element_type=jnp.float32)
    xn = xn + bn2_ref[...]
    xn_ref[...] = xn

    part = jnp.dot(oht, xn, preferred_element_type=jnp.float32)

    @pl.when(i == 0)
    def _():
        nagg_ref[...] = jnp.zeros_like(nagg_ref)

    nagg_ref[...] += part

    @pl.when(i == nsteps - 1)
    def _():
        nagg = nagg_ref[...]
        eagg = ea_ref[...]
        gpre = jnp.dot(nagg, wga_ref[...], preferred_element_type=jnp.float32)
        gpre = gpre + jnp.dot(eagg, wgb_ref[...],
                              preferred_element_type=jnp.float32)
        gpre = gpre + jnp.dot(u, wgc_ref[...], preferred_element_type=jnp.float32)
        gpre = gpre + bg1_ref[...]
        gh = jnp.maximum(gpre, 0.0)
        un = u + jnp.dot(gh, wg2_ref[...], preferred_element_type=jnp.float32)
        un_ref[...] = un + bg2_ref[...]


def _run_node(x, pa, batch3, eagg, u,
              wna, wnb, wnc, bn1, wn2, bn2,
              wga, wgb, wgc, bg1, wg2, bg2):
    nsteps = N // BLK_N
    full = lambda r, c: pl.BlockSpec((r, c), lambda i: (0, 0))
    return pl.pallas_call(
        _node_body,
        grid=(nsteps,),
        in_specs=[
            pl.BlockSpec((BLK_N, D), lambda i: (i, 0)),
            pl.BlockSpec((1, BLK_N, H), lambda i: (0, i, 0)),
            pl.BlockSpec((1, BLK_N, H), lambda i: (1, i, 0)),
            pl.BlockSpec((1, 1, BLK_N), lambda i: (i, 0, 0)),
            full(G, D), full(G, D),
            full(D, H), full(D, H), full(D, H), full(1, H),
            full(H, D), full(1, D),
            full(D, H), full(D, H), full(D, H), full(1, H),
            full(H, D), full(1, D),
        ],
        out_specs=[
            pl.BlockSpec((BLK_N, D), lambda i: (i, 0)),
            pl.BlockSpec((G, D), lambda i: (0, 0)),
        ],
        out_shape=[
            jax.ShapeDtypeStruct((N, D), jnp.float32),
            jax.ShapeDtypeStruct((G, D), jnp.float32),
        ],
        scratch_shapes=[pltpu.VMEM((G, D), jnp.float32)],
    )(x, pa, pa, batch3, eagg, u,
      wna, wnb, wnc, bn1, wn2, bn2,
      wga, wgb, wgc, bg1, wg2, bg2)


def kernel(x, edge_attr, edge_index, batch, u, We1, be1, We2, be2,
           Wn1, bn1, Wn2, bn2, Wg1, bg1, Wg2, bg2):
    src = edge_index[0].astype(jnp.int32)
    dst = edge_index[1].astype(jnp.int32)
    batch_i = batch.astype(jnp.int32)

    src2 = src.reshape(E // CH, CH)
    dst2 = dst.reshape(E // CH, CH)
    nslab_s = NPT_S * NW
    src3f = src2[:nslab_s].reshape(NW, NPT_S, CH)
    srcxf = src2[nslab_s:].reshape(NEXTRA_S, 1, CH)
    dst3f = dst2[:nslab_s].reshape(NW, NPT_S, CH)
    dstxf = dst2[nslab_s:].reshape(NEXTRA_S, 1, CH)

    batch3 = batch_i.astype(jnp.float32).reshape(N // BLK_N, 1, BLK_N)

    wa = We1[:D]
    wb = We1[D:2 * D]
    wc = We1[2 * D:3 * D]
    wd = We1[3 * D:]
    be1r = be1.reshape(1, H)
    be2r = be2.reshape(1, D)
    wna, wnb, wnc = Wn1[:D], Wn1[D:2 * D], Wn1[2 * D:]
    bn1r = bn1.reshape(1, H)
    bn2r = bn2.reshape(1, D)
    wga, wgb, wgc = Wg1[:D], Wg1[D:2 * D], Wg1[2 * D:]
    bg1r = bg1.reshape(1, H)
    bg2r = bg2.reshape(1, D)
    zin = jnp.zeros((ZROWS, H), jnp.float32)

    a2, bm = _run_prep(x, batch3, u, wa, wb, wd, be1r)

    g, bs = _run_sc_gather(a2, bm, batch_i, src3f, dst3f, srcxf, dstxf)

    bs3 = bs.reshape(E // BLK_E, 1, BLK_E)

    edge_new, eagg = _run_edge(edge_attr, g, bs3, wc, We2, be2r)

    parts = _run_sc_scatter(edge_new, dst3f, dstxf, zin)

    x_new, u_new = _run_node(
        x, parts, batch3, eagg, u,
        wna, wnb, wnc, bn1r, Wn2, bn2r,
        wga, wgb, wgc, bg1r, Wg2, bg2r)

    return (x_new, edge_new, u_new)
